# baseline (speedup 1.0000x reference)
import functools

import jax
import jax.numpy as jnp
from jax.experimental import pallas as pl
from jax.experimental.pallas import tpu as pltpu

_B, _S, _H, _F, _E, _K = 2, 2048, 1024, 4096, 8, 2
_N = _B * _S
_TM = 256
_PAD = _N * _K + _E * _TM
_NT = _PAD // _TM
_HT = _NT // 2
_TF = 512
_NF = _F // _TF
_TC = 256
_RB = 4


def _incl_cumsum(m, axis):
    x = m
    k = 1
    while k < m.shape[axis]:
        if axis == 0:
            shifted = jnp.concatenate(
                [jnp.zeros((k, x.shape[1]), x.dtype), x[:-k, :]], axis=0)
        else:
            shifted = jnp.concatenate(
                [jnp.zeros((x.shape[0], k), x.dtype), x[:, :-k]], axis=1)
        x = x + shifted
        k *= 2
    return x


def _router_kernel(x_ref, wr_ref, combine_w_ref, drows_ref, te_ref, loss_ref):
    x = x_ref[...]
    logits = jnp.dot(x, wr_ref[...], preferred_element_type=jnp.float32,
                     precision=jax.lax.Precision.DEFAULT)
    idx = jax.lax.broadcasted_iota(jnp.int32, logits.shape, 1)
    m1 = jnp.max(logits, axis=-1, keepdims=True)
    i1 = jnp.min(jnp.where(logits == m1, idx, _E), axis=-1, keepdims=True)
    mask1 = idx == i1
    l2 = jnp.where(mask1, -jnp.inf, logits)
    m2 = jnp.max(l2, axis=-1, keepdims=True)
    i2 = jnp.min(jnp.where(l2 == m2, idx, _E), axis=-1, keepdims=True)
    mask2 = idx == i2
    probs = jax.nn.softmax(logits, axis=-1)
    p1 = jnp.sum(jnp.where(mask1, probs, 0.0), axis=-1, keepdims=True)
    p2 = jnp.sum(jnp.where(mask2, probs, 0.0), axis=-1, keepdims=True)
    s = p1 + p2
    combine_w_ref[...] = jnp.concatenate([p1 / s, p2 / s], axis=1)

    routed = (mask1 | mask2).astype(jnp.float32)
    f = jnp.mean(routed, axis=0, keepdims=True)
    p = jnp.mean(probs, axis=0, keepdims=True)
    loss_ref[...] = (_E * jnp.sum(f * p)).reshape(1, 1)

    m1f = mask1.astype(jnp.float32)
    m2f = mask2.astype(jnp.float32)
    cum0 = _incl_cumsum(m1f, 0) - m1f
    cum1 = _incl_cumsum(m2f, 0) - m2f
    cnt0 = jnp.sum(m1f, axis=0, keepdims=True)
    counts = cnt0 + jnp.sum(m2f, axis=0, keepdims=True)
    tiles_e = jnp.ceil(counts / _TM)
    padded = tiles_e * _TM
    off = _incl_cumsum(padded, 1) - padded
    rank0 = jnp.sum(cum0 * m1f, axis=1, keepdims=True)
    rank1 = jnp.sum((cnt0 + cum1) * m2f, axis=1, keepdims=True)
    d0 = jnp.sum(off * m1f, axis=1, keepdims=True) + rank0
    d1 = jnp.sum(off * m2f, axis=1, keepdims=True) + rank1
    drows_ref[...] = jnp.concatenate([d0, d1], axis=1).astype(jnp.int32)

    tile_end = (off + padded) / _TM
    mm = jax.lax.broadcasted_iota(jnp.int32, (64, _E), 0).astype(jnp.float32)
    te = jnp.sum((mm >= tile_end).astype(jnp.int32), axis=1, keepdims=True)
    te_ref[...] = jnp.minimum(te, _E - 1)


def _dispatch_kernel(d0r_ref, d1r_ref, x_ref, xd_ref, xbf_ref):
    m = pl.program_id(0)

    @pl.when(m == 0)
    def _load_x():
        xbf_ref[...] = x_ref[...].astype(jnp.bfloat16)

    r = jax.lax.broadcasted_iota(jnp.int32, (_TM, _N), 0) + m * _TM
    hit = (d0r_ref[...] == r) | (d1r_ref[...] == r)
    perm = hit.astype(jnp.bfloat16)
    xd_ref[...] = jnp.dot(perm, xbf_ref[...],
                          preferred_element_type=jnp.float32).astype(jnp.bfloat16)


def _ffn_kernel(te_ref, xd_ref, wg_ref, wu_ref, wd_ref, yd_ref,
                acc_ref, wgb_ref, wub_ref, wdb_ref, *, off):
    f = pl.program_id(0)
    m = pl.program_id(1)

    gm = m + off
    changed = (m == 0) | (te_ref[gm] != te_ref[jnp.maximum(gm - 1, 0)])

    @pl.when(changed)
    def _recast():
        wgb_ref[...] = wg_ref[0].astype(jnp.bfloat16)
        wub_ref[...] = wu_ref[0].astype(jnp.bfloat16)
        wdb_ref[...] = wd_ref[0].astype(jnp.bfloat16)

    xb = xd_ref[...]
    g = jnp.dot(xb, wgb_ref[...], preferred_element_type=jnp.float32)
    u = jnp.dot(xb, wub_ref[...], preferred_element_type=jnp.float32)
    h = (jax.nn.silu(g) * u).astype(jnp.bfloat16)
    part = jnp.dot(h, wdb_ref[...], preferred_element_type=jnp.float32)
    sl = (pl.ds(m * _TM, _TM), slice(None))

    @pl.when(f == 0)
    def _first():
        acc_ref[sl] = part

    @pl.when(f > 0)
    def _rest():
        acc_ref[sl] += part

    @pl.when(f == _NF - 1)
    def _fin():
        yd_ref[...] = acc_ref[sl].astype(jnp.bfloat16)


def _combine_kernel(d0c_ref, d1c_ref, w_ref, yd_ref, out_ref, acc_ref):
    rblk = pl.program_id(1)
    rc = _PAD // _RB
    r = jax.lax.broadcasted_iota(jnp.int32, (_TC, rc), 1) + rblk * rc
    w = w_ref[...]
    c = (jnp.where(d0c_ref[...] == r, w[:, 0:1], 0.0)
         + jnp.where(d1c_ref[...] == r, w[:, 1:2], 0.0)).astype(jnp.bfloat16)
    part = jnp.dot(c, yd_ref[...], preferred_element_type=jnp.float32)

    @pl.when(rblk == 0)
    def _init():
        acc_ref[...] = jnp.zeros_like(acc_ref)

    acc_ref[...] += part

    @pl.when(rblk == _RB - 1)
    def _fin():
        out_ref[...] = acc_ref[...]


def _ffn_half(te1d, xd, Wg, Wu, Wd, off):
    return pl.pallas_call(
        functools.partial(_ffn_kernel, off=off),
        grid_spec=pltpu.PrefetchScalarGridSpec(
            num_scalar_prefetch=1,
            grid=(_NF, _HT),
            in_specs=[
                pl.BlockSpec((_TM, _H), lambda f, m, te: (m + off, 0)),
                pl.BlockSpec((1, _H, _TF), lambda f, m, te: (te[m + off], 0, f)),
                pl.BlockSpec((1, _H, _TF), lambda f, m, te: (te[m + off], 0, f)),
                pl.BlockSpec((1, _TF, _H), lambda f, m, te: (te[m + off], f, 0)),
            ],
            out_specs=pl.BlockSpec((_TM, _H), lambda f, m, te: (m, 0)),
            scratch_shapes=[
                pltpu.VMEM((_HT * _TM, _H), jnp.float32),
                pltpu.VMEM((_H, _TF), jnp.bfloat16),
                pltpu.VMEM((_H, _TF), jnp.bfloat16),
                pltpu.VMEM((_TF, _H), jnp.bfloat16),
            ],
        ),
        out_shape=jax.ShapeDtypeStruct((_HT * _TM, _H), jnp.bfloat16),
        compiler_params=pltpu.CompilerParams(
            dimension_semantics=("arbitrary", "arbitrary"),
        ),
    )(te1d, xd, Wg, Wu, Wd)


def kernel(hidden_states, Wr, Wg, Wu, Wd):
    x2d = hidden_states.reshape(_N, _H)

    combine_w, drows, te, loss = pl.pallas_call(
        _router_kernel,
        out_shape=(
            jax.ShapeDtypeStruct((_N, _K), jnp.float32),
            jax.ShapeDtypeStruct((_N, _K), jnp.int32),
            jax.ShapeDtypeStruct((64, 1), jnp.int32),
            jax.ShapeDtypeStruct((1, 1), jnp.float32),
        ),
    )(x2d, Wr)

    te1d = te.reshape(64)
    d0r = drows[:, 0:1].reshape(1, _N)
    d1r = drows[:, 1:2].reshape(1, _N)

    xd = pl.pallas_call(
        _dispatch_kernel,
        grid=(_NT,),
        in_specs=[
            pl.BlockSpec((1, _N), lambda m: (0, 0)),
            pl.BlockSpec((1, _N), lambda m: (0, 0)),
            pl.BlockSpec((_N, _H), lambda m: (0, 0)),
        ],
        out_specs=pl.BlockSpec((_TM, _H), lambda m: (m, 0)),
        out_shape=jax.ShapeDtypeStruct((_PAD, _H), jnp.bfloat16),
        scratch_shapes=[pltpu.VMEM((_N, _H), jnp.bfloat16)],
    )(d0r, d1r, x2d)

    yd_a = _ffn_half(te1d, xd, Wg, Wu, Wd, 0)
    yd_b = _ffn_half(te1d, xd, Wg, Wu, Wd, _HT)
    yd = jnp.concatenate([yd_a, yd_b], axis=0)

    d0c = drows[:, 0:1]
    d1c = drows[:, 1:2]
    out = pl.pallas_call(
        _combine_kernel,
        grid=(_N // _TC, _RB),
        in_specs=[
            pl.BlockSpec((_TC, 1), lambda t, r: (t, 0)),
            pl.BlockSpec((_TC, 1), lambda t, r: (t, 0)),
            pl.BlockSpec((_TC, _K), lambda t, r: (t, 0)),
            pl.BlockSpec((_PAD // _RB, _H), lambda t, r: (r, 0)),
        ],
        out_specs=pl.BlockSpec((_TC, _H), lambda t, r: (t, 0)),
        out_shape=jax.ShapeDtypeStruct((_N, _H), jnp.float32),
        scratch_shapes=[pltpu.VMEM((_TC, _H), jnp.float32)],
        compiler_params=pltpu.CompilerParams(
            dimension_semantics=("parallel", "arbitrary"),
        ),
    )(d0c, d1c, combine_w, yd)

    return out.reshape(_B, _S, _H), loss[0, 0]


# speedup vs baseline: 1.0697x; 1.0697x over previous
import functools

import jax
import jax.numpy as jnp
from jax.experimental import pallas as pl
from jax.experimental.pallas import tpu as pltpu

_B, _S, _H, _F, _E, _K = 2, 2048, 1024, 4096, 8, 2
_N = _B * _S
_TM = 256
_PAD = _N * _K + _E * _TM
_NT = _PAD // _TM
_HT = _NT // 2
_TF = 512
_NF = _F // _TF
_TC = 256
_RB = 4


def _incl_cumsum(m, axis):
    x = m
    k = 1
    while k < m.shape[axis]:
        if axis == 0:
            shifted = jnp.concatenate(
                [jnp.zeros((k, x.shape[1]), x.dtype), x[:-k, :]], axis=0)
        else:
            shifted = jnp.concatenate(
                [jnp.zeros((x.shape[0], k), x.dtype), x[:, :-k]], axis=1)
        x = x + shifted
        k *= 2
    return x


def _router_kernel(x_ref, wr_ref, combine_w_ref, drows_ref, te_ref, loss_ref):
    x = x_ref[...]
    logits = jnp.dot(x, wr_ref[...], preferred_element_type=jnp.float32,
                     precision=jax.lax.Precision.DEFAULT)
    idx = jax.lax.broadcasted_iota(jnp.int32, logits.shape, 1)
    m1 = jnp.max(logits, axis=-1, keepdims=True)
    i1 = jnp.min(jnp.where(logits == m1, idx, _E), axis=-1, keepdims=True)
    mask1 = idx == i1
    l2 = jnp.where(mask1, -jnp.inf, logits)
    m2 = jnp.max(l2, axis=-1, keepdims=True)
    i2 = jnp.min(jnp.where(l2 == m2, idx, _E), axis=-1, keepdims=True)
    mask2 = idx == i2
    probs = jax.nn.softmax(logits, axis=-1)
    p1 = jnp.sum(jnp.where(mask1, probs, 0.0), axis=-1, keepdims=True)
    p2 = jnp.sum(jnp.where(mask2, probs, 0.0), axis=-1, keepdims=True)
    s = p1 + p2
    combine_w_ref[...] = jnp.concatenate([p1 / s, p2 / s], axis=1)

    routed = (mask1 | mask2).astype(jnp.float32)
    f = jnp.mean(routed, axis=0, keepdims=True)
    p = jnp.mean(probs, axis=0, keepdims=True)
    loss_ref[...] = (_E * jnp.sum(f * p)).reshape(1, 1)

    m1f = mask1.astype(jnp.float32)
    m2f = mask2.astype(jnp.float32)
    cum0 = _incl_cumsum(m1f, 0) - m1f
    cum1 = _incl_cumsum(m2f, 0) - m2f
    cnt0 = jnp.sum(m1f, axis=0, keepdims=True)
    counts = cnt0 + jnp.sum(m2f, axis=0, keepdims=True)
    tiles_e = jnp.ceil(counts / _TM)
    padded = tiles_e * _TM
    off = _incl_cumsum(padded, 1) - padded
    rank0 = jnp.sum(cum0 * m1f, axis=1, keepdims=True)
    rank1 = jnp.sum((cnt0 + cum1) * m2f, axis=1, keepdims=True)
    d0 = jnp.sum(off * m1f, axis=1, keepdims=True) + rank0
    d1 = jnp.sum(off * m2f, axis=1, keepdims=True) + rank1
    drows_ref[...] = jnp.concatenate([d0, d1], axis=1).astype(jnp.int32)

    tile_end = (off + padded) / _TM
    mm = jax.lax.broadcasted_iota(jnp.int32, (64, _E), 0).astype(jnp.float32)
    te = jnp.sum((mm >= tile_end).astype(jnp.int32), axis=1, keepdims=True)
    te_ref[...] = jnp.minimum(te, _E - 1)


def _dispatch_kernel(d0r_ref, d1r_ref, x_ref, xd_ref, xbf_ref):
    m = pl.program_id(0)

    @pl.when(m == 0)
    def _load_x():
        xbf_ref[...] = x_ref[...].astype(jnp.bfloat16)

    r = jax.lax.broadcasted_iota(jnp.int32, (_TM, _N), 0) + m * _TM
    hit = (d0r_ref[...] == r) | (d1r_ref[...] == r)
    perm = hit.astype(jnp.bfloat16)
    xd_ref[...] = jnp.dot(perm, xbf_ref[...],
                          preferred_element_type=jnp.float32).astype(jnp.bfloat16)


def _ffn_kernel(te_ref, xd_ref, wg_ref, wu_ref, wd_ref, yd_ref,
                acc_ref, wgb_ref, wub_ref, wdb_ref):
    f = pl.program_id(0)
    m = pl.program_id(1)

    changed = (m == 0) | (te_ref[m] != te_ref[jnp.maximum(m - 1, 0)])

    @pl.when(changed)
    def _recast():
        wgb_ref[...] = wg_ref[0].astype(jnp.bfloat16)
        wub_ref[...] = wu_ref[0].astype(jnp.bfloat16)
        wdb_ref[...] = wd_ref[0].astype(jnp.bfloat16)

    xb = xd_ref[...]
    g = jnp.dot(xb, wgb_ref[...], preferred_element_type=jnp.float32)
    u = jnp.dot(xb, wub_ref[...], preferred_element_type=jnp.float32)
    h = (jax.nn.silu(g) * u).astype(jnp.bfloat16)
    part = jnp.dot(h, wdb_ref[...], preferred_element_type=jnp.float32)
    sl = (pl.ds(m * _TM, _TM), slice(None))
    acc = jnp.where(f == 0, part, acc_ref[sl] + part)
    acc_ref[sl] = acc
    yd_ref[...] = acc.astype(jnp.bfloat16)


def _combine_kernel(d0c_ref, d1c_ref, w_ref, yd_ref, out_ref, acc_ref):
    rblk = pl.program_id(1)
    rc = _PAD // _RB
    r = jax.lax.broadcasted_iota(jnp.int32, (_TC, rc), 1) + rblk * rc
    w = w_ref[...]
    c = (jnp.where(d0c_ref[...] == r, w[:, 0:1], 0.0)
         + jnp.where(d1c_ref[...] == r, w[:, 1:2], 0.0)).astype(jnp.bfloat16)
    part = jnp.dot(c, yd_ref[...], preferred_element_type=jnp.float32)

    @pl.when(rblk == 0)
    def _init():
        acc_ref[...] = jnp.zeros_like(acc_ref)

    acc_ref[...] += part

    @pl.when(rblk == _RB - 1)
    def _fin():
        out_ref[...] = acc_ref[...]


def _ffn_call(te1d, xd, Wg, Wu, Wd):
    return pl.pallas_call(
        _ffn_kernel,
        grid_spec=pltpu.PrefetchScalarGridSpec(
            num_scalar_prefetch=1,
            grid=(_NF, _NT),
            in_specs=[
                pl.BlockSpec((_TM, _H), lambda f, m, te: (m, 0)),
                pl.BlockSpec((1, _H, _TF), lambda f, m, te: (te[m], 0, f)),
                pl.BlockSpec((1, _H, _TF), lambda f, m, te: (te[m], 0, f)),
                pl.BlockSpec((1, _TF, _H), lambda f, m, te: (te[m], f, 0)),
            ],
            out_specs=pl.BlockSpec((_TM, _H), lambda f, m, te: (m, 0)),
            scratch_shapes=[
                pltpu.VMEM((_PAD, _H), jnp.float32),
                pltpu.VMEM((_H, _TF), jnp.bfloat16),
                pltpu.VMEM((_H, _TF), jnp.bfloat16),
                pltpu.VMEM((_TF, _H), jnp.bfloat16),
            ],
        ),
        out_shape=jax.ShapeDtypeStruct((_PAD, _H), jnp.bfloat16),
        compiler_params=pltpu.CompilerParams(
            dimension_semantics=("arbitrary", "arbitrary"),
        ),
    )(te1d, xd, Wg, Wu, Wd)


def kernel(hidden_states, Wr, Wg, Wu, Wd):
    x2d = hidden_states.reshape(_N, _H)

    combine_w, drows, te, loss = pl.pallas_call(
        _router_kernel,
        out_shape=(
            jax.ShapeDtypeStruct((_N, _K), jnp.float32),
            jax.ShapeDtypeStruct((_N, _K), jnp.int32),
            jax.ShapeDtypeStruct((64, 1), jnp.int32),
            jax.ShapeDtypeStruct((1, 1), jnp.float32),
        ),
    )(x2d, Wr)

    te1d = te.reshape(64)
    d0r = drows[:, 0:1].reshape(1, _N)
    d1r = drows[:, 1:2].reshape(1, _N)

    xd = pl.pallas_call(
        _dispatch_kernel,
        grid=(_NT,),
        in_specs=[
            pl.BlockSpec((1, _N), lambda m: (0, 0)),
            pl.BlockSpec((1, _N), lambda m: (0, 0)),
            pl.BlockSpec((_N, _H), lambda m: (0, 0)),
        ],
        out_specs=pl.BlockSpec((_TM, _H), lambda m: (m, 0)),
        out_shape=jax.ShapeDtypeStruct((_PAD, _H), jnp.bfloat16),
        scratch_shapes=[pltpu.VMEM((_N, _H), jnp.bfloat16)],
    )(d0r, d1r, x2d)

    yd = _ffn_call(te1d, xd, Wg, Wu, Wd)

    d0c = drows[:, 0:1]
    d1c = drows[:, 1:2]
    out = pl.pallas_call(
        _combine_kernel,
        grid=(_N // _TC, _RB),
        in_specs=[
            pl.BlockSpec((_TC, 1), lambda t, r: (t, 0)),
            pl.BlockSpec((_TC, 1), lambda t, r: (t, 0)),
            pl.BlockSpec((_TC, _K), lambda t, r: (t, 0)),
            pl.BlockSpec((_PAD // _RB, _H), lambda t, r: (r, 0)),
        ],
        out_specs=pl.BlockSpec((_TC, _H), lambda t, r: (t, 0)),
        out_shape=jax.ShapeDtypeStruct((_N, _H), jnp.float32),
        scratch_shapes=[pltpu.VMEM((_TC, _H), jnp.float32)],
        compiler_params=pltpu.CompilerParams(
            dimension_semantics=("parallel", "arbitrary"),
        ),
    )(d0c, d1c, combine_w, yd)

    return out.reshape(_B, _S, _H), loss[0, 0]


# speedup vs baseline: 1.1636x; 1.0877x over previous
import functools

import jax
import jax.numpy as jnp
from jax.experimental import pallas as pl
from jax.experimental.pallas import tpu as pltpu
from jax.experimental.pallas import tpu_sc as plsc

_B, _S, _H, _F, _E, _K = 2, 2048, 1024, 4096, 8, 2
_N = _B * _S
_TM = 256
_PAD = _N * _K + _E * _TM
_NT = _PAD // _TM
_HT = _NT // 2
_TF = 512
_NF = _F // _TF
_TC = 256
_RB = 4


def _incl_cumsum(m, axis):
    x = m
    k = 1
    while k < m.shape[axis]:
        if axis == 0:
            shifted = jnp.concatenate(
                [jnp.zeros((k, x.shape[1]), x.dtype), x[:-k, :]], axis=0)
        else:
            shifted = jnp.concatenate(
                [jnp.zeros((x.shape[0], k), x.dtype), x[:, :-k]], axis=1)
        x = x + shifted
        k *= 2
    return x


def _router_kernel(x_ref, wr_ref, combine_w_ref, drows_ref, te_ref, loss_ref):
    x = x_ref[...]
    logits = jnp.dot(x, wr_ref[...], preferred_element_type=jnp.float32,
                     precision=jax.lax.Precision.DEFAULT)
    idx = jax.lax.broadcasted_iota(jnp.int32, logits.shape, 1)
    m1 = jnp.max(logits, axis=-1, keepdims=True)
    i1 = jnp.min(jnp.where(logits == m1, idx, _E), axis=-1, keepdims=True)
    mask1 = idx == i1
    l2 = jnp.where(mask1, -jnp.inf, logits)
    m2 = jnp.max(l2, axis=-1, keepdims=True)
    i2 = jnp.min(jnp.where(l2 == m2, idx, _E), axis=-1, keepdims=True)
    mask2 = idx == i2
    probs = jax.nn.softmax(logits, axis=-1)
    p1 = jnp.sum(jnp.where(mask1, probs, 0.0), axis=-1, keepdims=True)
    p2 = jnp.sum(jnp.where(mask2, probs, 0.0), axis=-1, keepdims=True)
    s = p1 + p2
    combine_w_ref[...] = jnp.concatenate([p1 / s, p2 / s], axis=1)

    routed = (mask1 | mask2).astype(jnp.float32)
    f = jnp.mean(routed, axis=0, keepdims=True)
    p = jnp.mean(probs, axis=0, keepdims=True)
    loss_ref[...] = (_E * jnp.sum(f * p)).reshape(1, 1)

    m1f = mask1.astype(jnp.float32)
    m2f = mask2.astype(jnp.float32)
    cum0 = _incl_cumsum(m1f, 0) - m1f
    cum1 = _incl_cumsum(m2f, 0) - m2f
    cnt0 = jnp.sum(m1f, axis=0, keepdims=True)
    counts = cnt0 + jnp.sum(m2f, axis=0, keepdims=True)
    tiles_e = jnp.ceil(counts / _TM)
    padded = tiles_e * _TM
    off = _incl_cumsum(padded, 1) - padded
    rank0 = jnp.sum(cum0 * m1f, axis=1, keepdims=True)
    rank1 = jnp.sum((cnt0 + cum1) * m2f, axis=1, keepdims=True)
    d0 = jnp.sum(off * m1f, axis=1, keepdims=True) + rank0
    d1 = jnp.sum(off * m2f, axis=1, keepdims=True) + rank1
    drows_ref[...] = jnp.concatenate([d0, d1], axis=1).astype(jnp.int32)

    tile_end = (off + padded) / _TM
    mm = jax.lax.broadcasted_iota(jnp.int32, (64, _E), 0).astype(jnp.float32)
    te = jnp.sum((mm >= tile_end).astype(jnp.int32), axis=1, keepdims=True)
    te_ref[...] = jnp.minimum(te, _E - 1)


_SCW = 128


def _sc_dispatch(xbf, dall):
    nxb = _N // _SCW

    hc = _H // 4

    @pl.kernel(
        out_type=jax.ShapeDtypeStruct((_PAD, _H), jnp.float32),
        mesh=plsc.VectorSubcoreMesh(core_axis_name="c", subcore_axis_name="s"),
    )
    def sc_kernel(x_hbm, i_hbm, xd_hbm):
        for c in range(4):
            def body(x_vmem, i_vmem, c=c):
                pltpu.sync_copy(
                    x_vmem, xd_hbm.at[i_vmem.at[0], pl.ds(c * hc, hc)])

            pltpu.emit_pipeline(
                body,
                grid=(2 * nxb,),
                in_specs=[
                    pl.BlockSpec((_SCW, hc),
                                 index_map=lambda i, c=c: (i % nxb, c)),
                    pl.BlockSpec((1, _SCW), index_map=lambda i: (0, i)),
                ],
                out_specs=[],
                core_axis_name=("c", "s"),
                dimension_semantics=(pltpu.PARALLEL,),
            )(x_hbm, i_hbm)

    return sc_kernel(xbf, dall)


def _ffn_kernel(te_ref, xd_ref, wg_ref, wu_ref, wd_ref, yd_ref, acc_ref):
    f = pl.program_id(0)
    m = pl.program_id(1)
    xb = xd_ref[...].astype(jnp.bfloat16)
    g = jnp.dot(xb, wg_ref[0].astype(jnp.bfloat16),
                preferred_element_type=jnp.float32)
    u = jnp.dot(xb, wu_ref[0].astype(jnp.bfloat16),
                preferred_element_type=jnp.float32)
    h = (jax.nn.silu(g) * u).astype(jnp.bfloat16)
    part = jnp.dot(h, wd_ref[0].astype(jnp.bfloat16),
                   preferred_element_type=jnp.float32)
    sl = (pl.ds(m * _TM, _TM), slice(None))
    acc = jnp.where(f == 0, part, acc_ref[sl] + part)
    acc_ref[sl] = acc
    yd_ref[...] = acc.astype(jnp.bfloat16)


def _combine_kernel(d0c_ref, d1c_ref, w_ref, yd_ref, out_ref, acc_ref):
    rblk = pl.program_id(1)
    rc = _PAD // _RB
    r = jax.lax.broadcasted_iota(jnp.int32, (_TC, rc), 1) + rblk * rc
    w = w_ref[...]
    c = (jnp.where(d0c_ref[...] == r, w[:, 0:1], 0.0)
         + jnp.where(d1c_ref[...] == r, w[:, 1:2], 0.0)).astype(jnp.bfloat16)
    part = jnp.dot(c, yd_ref[...], preferred_element_type=jnp.float32)

    @pl.when(rblk == 0)
    def _init():
        acc_ref[...] = jnp.zeros_like(acc_ref)

    acc_ref[...] += part

    @pl.when(rblk == _RB - 1)
    def _fin():
        out_ref[...] = acc_ref[...]


def _ffn_call(te1d, xd, Wg, Wu, Wd):
    return pl.pallas_call(
        _ffn_kernel,
        grid_spec=pltpu.PrefetchScalarGridSpec(
            num_scalar_prefetch=1,
            grid=(_NF, _NT),
            in_specs=[
                pl.BlockSpec((_TM, _H), lambda f, m, te: (m, 0)),
                pl.BlockSpec((1, _H, _TF), lambda f, m, te: (te[m], 0, f)),
                pl.BlockSpec((1, _H, _TF), lambda f, m, te: (te[m], 0, f)),
                pl.BlockSpec((1, _TF, _H), lambda f, m, te: (te[m], f, 0)),
            ],
            out_specs=pl.BlockSpec((_TM, _H), lambda f, m, te: (m, 0)),
            scratch_shapes=[pltpu.VMEM((_PAD, _H), jnp.float32)],
        ),
        out_shape=jax.ShapeDtypeStruct((_PAD, _H), jnp.bfloat16),
        compiler_params=pltpu.CompilerParams(
            dimension_semantics=("arbitrary", "arbitrary"),
        ),
    )(te1d, xd, Wg, Wu, Wd)


def kernel(hidden_states, Wr, Wg, Wu, Wd):
    x2d = hidden_states.reshape(_N, _H)

    combine_w, drows, te, loss = pl.pallas_call(
        _router_kernel,
        out_shape=(
            jax.ShapeDtypeStruct((_N, _K), jnp.float32),
            jax.ShapeDtypeStruct((_N, _K), jnp.int32),
            jax.ShapeDtypeStruct((64, 1), jnp.int32),
            jax.ShapeDtypeStruct((1, 1), jnp.float32),
        ),
    )(x2d, Wr)

    te1d = te.reshape(64)
    dall = jnp.transpose(drows, (1, 0)).reshape(1, 2 * _N)
    xd = _sc_dispatch(x2d, dall)

    yd = _ffn_call(te1d, xd, Wg, Wu, Wd)

    d0c = drows[:, 0:1]
    d1c = drows[:, 1:2]
    out = pl.pallas_call(
        _combine_kernel,
        grid=(_N // _TC, _RB),
        in_specs=[
            pl.BlockSpec((_TC, 1), lambda t, r: (t, 0)),
            pl.BlockSpec((_TC, 1), lambda t, r: (t, 0)),
            pl.BlockSpec((_TC, _K), lambda t, r: (t, 0)),
            pl.BlockSpec((_PAD // _RB, _H), lambda t, r: (r, 0)),
        ],
        out_specs=pl.BlockSpec((_TC, _H), lambda t, r: (t, 0)),
        out_shape=jax.ShapeDtypeStruct((_N, _H), jnp.float32),
        scratch_shapes=[pltpu.VMEM((_TC, _H), jnp.float32)],
        compiler_params=pltpu.CompilerParams(
            dimension_semantics=("parallel", "arbitrary"),
        ),
    )(d0c, d1c, combine_w, yd)

    return out.reshape(_B, _S, _H), loss[0, 0]
